# Initial kernel scaffold; baseline (speedup 1.0000x reference)
import jax
import jax.numpy as jnp
from jax.experimental import pallas as pl


def kernel(hidden_states, prior_norm_w, W_mu, W_logvar, dec_norm_w, dec_Wg, dec_Wu, dec_Wd, dyn_norm_w, dyn_Wg, dyn_Wu, dyn_Wd, w_cr, b_cr):
    raise NotImplementedError("write your pallas kernel here")


# speedup vs baseline: 1.4393x; 1.4393x over previous
import functools

import jax
import jax.numpy as jnp
from jax.experimental import pallas as pl

HIDDEN = 768
D_FF = 2048
EPS = 1e-06
INTERPRET = False


def _rms(x, w):
    v = jnp.mean(x * x, axis=-1, keepdims=True)
    return x * jax.lax.rsqrt(v + EPS) * w


def _k1_body(x_ref, decnw_ref, wg_ref, wu_ref, wd_ref, pnw_ref, wmu_ref,
             wlv_ref, wcr_ref, bcr_ref, proc_ref, g_ref, logit_ref, s1_ref):
    x = x_ref[...]
    h = _rms(x, decnw_ref[...])
    hg = jnp.dot(h, wg_ref[...], preferred_element_type=jnp.float32)
    hu = jnp.dot(h, wu_ref[...], preferred_element_type=jnp.float32)
    act = (hg * jax.nn.sigmoid(hg)) * hu
    delta = jnp.dot(act, wd_ref[...], preferred_element_type=jnp.float32)
    proc_ref[...] = x + delta

    xn = _rms(x, pnw_ref[...])
    mu = jnp.dot(xn, wmu_ref[...], preferred_element_type=jnp.float32)
    lv = jnp.dot(xn, wlv_ref[...], preferred_element_type=jnp.float32)
    d_st = jnp.sum(delta * delta, axis=-1) / float(HIDDEN)
    d_ch = 0.5 * jnp.mean(
        lv + (1.0 + (delta - mu) ** 2) * jnp.exp(-lv) - 1.0, axis=-1)
    g_ref[0, 0, :] = jax.nn.sigmoid(d_st - d_ch)

    logit = jnp.dot(x, wcr_ref[...],
                    preferred_element_type=jnp.float32)[:, 0] + bcr_ref[0, :]
    logit_ref[0, 0, :] = logit
    spl = jnp.maximum(logit, 0.0) + jnp.log1p(jnp.exp(-jnp.abs(logit)))
    s1_ref[0, 0, :] = spl


def _k2_body(g_ref, logit_ref, s1_ref, mg_ref, loss_ref, *, k, n_tok):
    g = g_ref[...]
    gi = jax.lax.bitcast_convert_type(g, jnp.int32)

    def bs_step(i, lo):
        cand = lo | jnp.left_shift(jnp.int32(1), 30 - i)
        cnt = jnp.sum((gi >= cand).astype(jnp.int32), axis=1, keepdims=True)
        return jnp.where(cnt >= k, cand, lo)

    lo = jnp.zeros((g.shape[0], 1), jnp.int32)
    thr = jax.lax.fori_loop(0, 31, bs_step, lo)

    m_gt = gi > thr
    n_gt = jnp.sum(m_gt.astype(jnp.int32), axis=1, keepdims=True)
    r = k - n_gt
    m_eq = gi == thr
    c = m_eq.astype(jnp.int32)
    s = 1
    while s < g.shape[1]:
        c = c + jnp.concatenate(
            [jnp.zeros((g.shape[0], s), jnp.int32), c[:, :-s]], axis=1)
        s *= 2
    mask = m_gt | (m_eq & (c <= r))
    mg_ref[...] = mask.astype(jnp.float32) * g
    lsel = jnp.sum(jnp.where(mask, logit_ref[...], 0.0))
    loss = (jnp.sum(s1_ref[...]) - lsel) / float(n_tok)
    loss_ref[0, :] = jnp.full((128,), loss)


def _k3_body(p_ref, mg_ref, nw_ref, wg_ref, wu_ref, wd_ref, out_ref):
    p = p_ref[...]
    h = _rms(p, nw_ref[...])
    hg = jnp.dot(h, wg_ref[...], preferred_element_type=jnp.float32)
    hu = jnp.dot(h, wu_ref[...], preferred_element_type=jnp.float32)
    act = (hg * jax.nn.sigmoid(hg)) * hu
    delta = jnp.dot(act, wd_ref[...], preferred_element_type=jnp.float32)
    out_ref[...] = p + mg_ref[...].reshape(-1, 1) * delta


def kernel(hidden_states, prior_norm_w, W_mu, W_logvar, dec_norm_w, dec_Wg,
           dec_Wu, dec_Wd, dyn_norm_w, dyn_Wg, dyn_Wu, dyn_Wd, w_cr, b_cr):
    B, T, D = hidden_states.shape
    N = B * T
    BLK = 256
    nblk = N // BLK
    k = max(1, int(T * 0.25))

    x2d = hidden_states.reshape(N, D)
    row = lambda w: w.reshape(1, D)
    wcol = w_cr.reshape(D, 1)
    bcr = jnp.full((1, BLK), b_cr, jnp.float32)

    full = lambda shape: pl.BlockSpec(shape, lambda *_: (0,) * len(shape))
    tokb = pl.BlockSpec((BLK, D), lambda i: (i, 0))

    proc, g2, logit2, s1 = pl.pallas_call(
        _k1_body,
        grid=(nblk,),
        in_specs=[
            tokb, full((1, D)), full((D, D_FF)), full((D, D_FF)),
            full((D_FF, D)), full((1, D)), full((D, D)), full((D, D)),
            full((D, 1)), full((1, BLK)),
        ],
        out_specs=[
            tokb,
            pl.BlockSpec((1, 1, BLK), lambda i: (i, 0, 0)),
            pl.BlockSpec((1, 1, BLK), lambda i: (i, 0, 0)),
            pl.BlockSpec((1, 1, BLK), lambda i: (i, 0, 0)),
        ],
        out_shape=[
            jax.ShapeDtypeStruct((N, D), jnp.float32),
            jax.ShapeDtypeStruct((nblk, 1, BLK), jnp.float32),
            jax.ShapeDtypeStruct((nblk, 1, BLK), jnp.float32),
            jax.ShapeDtypeStruct((nblk, 1, BLK), jnp.float32),
        ],
        interpret=INTERPRET,
    )(x2d, row(dec_norm_w), dec_Wg, dec_Wu, dec_Wd, row(prior_norm_w), W_mu,
      W_logvar, wcol, bcr)

    g_bt = g2.reshape(B, T)
    logit_bt = logit2.reshape(B, T)

    mg, loss = pl.pallas_call(
        functools.partial(_k2_body, k=k, n_tok=N),
        in_specs=[full((B, T)), full((B, T)), full((nblk, 1, BLK))],
        out_specs=[full((B, T)), full((1, 128))],
        out_shape=[
            jax.ShapeDtypeStruct((B, T), jnp.float32),
            jax.ShapeDtypeStruct((1, 128), jnp.float32),
        ],
        interpret=INTERPRET,
    )(g_bt, logit_bt, s1)

    final2d = pl.pallas_call(
        _k3_body,
        grid=(nblk,),
        in_specs=[
            tokb, pl.BlockSpec((1, 1, BLK), lambda i: (i, 0, 0)), full((1, D)),
            full((D, D_FF)), full((D, D_FF)), full((D_FF, D)),
        ],
        out_specs=tokb,
        out_shape=jax.ShapeDtypeStruct((N, D), jnp.float32),
        interpret=INTERPRET,
    )(proc, mg.reshape(nblk, 1, BLK), row(dyn_norm_w), dyn_Wg, dyn_Wu, dyn_Wd)

    return (final2d.reshape(B, T, D), g_bt, loss[0, 0])
